# baseline (speedup 1.0000x reference)
import jax
import jax.numpy as jnp
from jax import lax
from jax.experimental import pallas as pl
from jax.experimental.pallas import tpu as pltpu
from jax.experimental.pallas import tpu_sc as plsc

_K_NN = 10
_TQ = 256
_RB_SC = 2048
_NC, _NS = 2, 16
_NW = _NC * _NS
_CAP = 96
_NVB = _CAP // 16
_SHRINK_AT = 48


def _mlp_body(x_ref, w1_ref, b1_ref, w2_ref, b2_ref, w3_ref, b3_ref,
              lns_ref, lnb_ref, out_ref):
    h = jax.nn.gelu(jnp.dot(x_ref[:], w1_ref[:]) + b1_ref[:])
    h = jax.nn.gelu(jnp.dot(h, w2_ref[:]) + b2_ref[:])
    h = jnp.dot(h, w3_ref[:]) + b3_ref[:]
    mu = jnp.mean(h, axis=-1, keepdims=True)
    var = jnp.var(h, axis=-1, keepdims=True)
    out_ref[:] = (h - mu) / jnp.sqrt(var + 1e-6) * lns_ref[:] + lnb_ref[:]


def _knn_body(q_ref, txT_ref, idx_ref, d_ref):
    q = q_ref[0]
    txT = txT_ref[0]
    acc = None
    for d in range(4):
        diff = q[:, d:d + 1] - txT[d:d + 1, :]
        sq = diff * diff
        acc = sq if acc is None else acc + sq

    n_ctx = acc.shape[1]
    colf = lax.broadcasted_iota(jnp.int32, acc.shape, 1).astype(jnp.float32)
    idx_cols, d_cols = [], []
    d2m = acc
    eqmask = None
    for _ in range(_K_NN):
        if eqmask is not None:
            d2m = jnp.where(eqmask, jnp.inf, d2m)
        m = jnp.min(d2m, axis=1, keepdims=True)
        eqmask = d2m == m
        posf = jnp.min(jnp.where(eqmask, colf, float(n_ctx)), axis=1,
                       keepdims=True)
        idx_cols.append(posf.astype(jnp.int32))
        d_cols.append(jnp.sqrt(jnp.maximum(m, 0.0)))
    idx_ref[0] = jnp.concatenate(idx_cols, axis=1)
    d_ref[0] = jnp.concatenate(d_cols, axis=1)


def _lane_min(v, iota16):
    for s in (1, 2, 4, 8):
        perm = jnp.bitwise_xor(iota16, s)
        v = jnp.minimum(v, v.at[perm].get(mode="promise_in_bounds"))
    return v


def _sc_knn_body(K, rows_w, qc_ref, txT_ref, idx_out, d2_out,
                 txv, qv, outi, outd, bufd, bufi):
    wid = lax.axis_index("s") * _NC + lax.axis_index("c")
    wpb = _NW // 4
    b = wid // wpb
    r0 = (wid % wpb) * rows_w
    pltpu.sync_copy(txT_ref.at[pl.ds(b * 4 * K, 4 * K)], txv)
    pltpu.sync_copy(qc_ref.at[pl.ds((b * _RB_SC + r0) * 4, rows_w * 4)],
                    qv.at[pl.ds(0, rows_w * 4)])
    iota16 = lax.iota(jnp.int32, 16)
    inf_vec = jnp.full((16,), jnp.inf, jnp.float32)
    bigi_vec = jnp.full((16,), 2 ** 30, jnp.int32)

    def row_fn(r, _):
        qvec = qv[pl.ds(r * 4, 16)]
        q0, q1, q2, q3 = qvec[0], qvec[1], qvec[2], qvec[3]
        for i in range(_NVB):
            bufd[pl.ds(i * 16, 16)] = inf_vec

        def scan_fn(v, carry):
            thresh, cnt = carry
            off = v * 16
            x0 = txv[pl.ds(off, 16)]
            x1 = txv[pl.ds(K + off, 16)]
            x2 = txv[pl.ds(2 * K + off, 16)]
            x3 = txv[pl.ds(3 * K + off, 16)]
            e0 = x0 - q0
            acc = e0 * e0
            e1 = x1 - q1
            acc = acc + e1 * e1
            e2 = x2 - q2
            acc = acc + e2 * e2
            e3 = x3 - q3
            acc = acc + e3 * e3
            m = acc <= thresh

            def append(op):
                t0, c0 = op
                mi = m.astype(jnp.int32)
                cs = plsc.cumsum(mi)
                posv = cs - 1 + c0
                msafe = m & (posv < _CAP)
                plsc.store_scatter(bufd, (posv,), acc, mask=msafe)
                plsc.store_scatter(bufi, (posv,), iota16 + off, mask=msafe)
                c1 = c0 + cs[15]

                def shrink(op2):
                    _t, c_in = op2
                    vs = [bufd[pl.ds(i * 16, 16)] for i in range(_NVB)]
                    bis = [bufi[pl.ds(i * 16, 16)] for i in range(_NVB)]
                    work = list(vs)
                    t = inf_vec
                    for _ in range(_K_NN):
                        mm = work[0]
                        for i in range(1, _NVB):
                            mm = jnp.minimum(mm, work[i])
                        t = _lane_min(mm, iota16)
                        work = [jnp.where(w == t, inf_vec, w) for w in work]
                    for i in range(_NVB):
                        bufd[pl.ds(i * 16, 16)] = inf_vec
                    c2 = jnp.zeros((16,), jnp.int32)
                    for i in range(_NVB):
                        keep = vs[i] <= t
                        ki = keep.astype(jnp.int32)
                        csk = plsc.cumsum(ki)
                        pk = csk - 1 + c2
                        plsc.store_scatter(bufd, (pk,), vs[i], mask=keep)
                        plsc.store_scatter(bufi, (pk,), bis[i], mask=keep)
                        c2 = c2 + csk[15]
                    return t, c2

                return lax.cond(c1[0] > _SHRINK_AT, shrink, lambda op2: op2,
                                (t0, c1))

            return lax.cond(jnp.any(m), append, lambda op: op,
                            (thresh, cnt))

        lax.fori_loop(0, K // 16, scan_fn,
                      (inf_vec, jnp.zeros((16,), jnp.int32)), unroll=8)

        vs = [bufd[pl.ds(i * 16, 16)] for i in range(_NVB)]
        bis = [bufi[pl.ds(i * 16, 16)] for i in range(_NVB)]
        work = list(vs)
        idxacc = jnp.zeros((16,), jnp.int32)
        dacc = inf_vec
        for kk in range(_K_NN):
            mm = work[0]
            for i in range(1, _NVB):
                mm = jnp.minimum(mm, work[i])
            mval = _lane_min(mm, iota16)
            iv = bigi_vec
            for i in range(_NVB):
                iv = jnp.minimum(iv, jnp.where(work[i] == mval, bis[i],
                                               bigi_vec))
            cidx = _lane_min(iv, iota16)
            work = [jnp.where((work[i] == mval) & (bis[i] == cidx), jnp.inf,
                              work[i]) for i in range(_NVB)]
            sel = iota16 == kk
            idxacc = jnp.where(sel, cidx, idxacc)
            dacc = jnp.where(sel, mval, dacc)
        outi[pl.ds(r * 16, 16)] = idxacc
        outd[pl.ds(r * 16, 16)] = dacc
        return 0

    lax.fori_loop(0, rows_w, row_fn, 0)
    base = (b * _RB_SC + r0) * 16
    pltpu.sync_copy(outi, idx_out.at[pl.ds(base, rows_w * 16)])
    pltpu.sync_copy(outd, d2_out.at[pl.ds(base, rows_w * 16)])


def kernel(s_ctx, f_ctx, s_test, embed_obs, W1, b1, W2, b2, W3, b3,
           ln_scale, ln_bias):
    k = _K_NN
    B, Q, d_s = s_test.shape
    K = s_ctx.shape[1]
    d_f = f_ctx.shape[-1]
    n_rows = Q + K
    rb = _RB_SC
    rows_w = B * rb // _NW

    Rq = jnp.concatenate([s_test, s_ctx], axis=1)
    txT = jnp.swapaxes(s_ctx, 1, 2)

    mesh = plsc.VectorSubcoreMesh(core_axis_name="c", subcore_axis_name="s",
                                  num_cores=_NC, num_subcores=_NS)
    sc_idx, sc_d2 = pl.kernel(
        lambda *refs: _sc_knn_body(K, rows_w, *refs),
        out_type=[
            jax.ShapeDtypeStruct((B * rb * 16,), jnp.int32),
            jax.ShapeDtypeStruct((B * rb * 16,), jnp.float32),
        ],
        mesh=mesh,
        compiler_params=pltpu.CompilerParams(needs_layout_passes=False),
        scratch_types=[
            pltpu.VMEM((4 * K,), jnp.float32),
            pltpu.VMEM((rows_w * 4 + 16,), jnp.float32),
            pltpu.VMEM((rows_w * 16,), jnp.int32),
            pltpu.VMEM((rows_w * 16,), jnp.float32),
            pltpu.VMEM((_CAP,), jnp.float32),
            pltpu.VMEM((_CAP,), jnp.int32),
        ],
    )(Rq[:, :rb].reshape(-1), txT.reshape(-1))
    idx_sc = sc_idx.reshape(B, rb, 16)[..., :k]
    d_sc = jnp.sqrt(jnp.maximum(sc_d2.reshape(B, rb, 16)[..., :k], 0.0))

    n_tc = n_rows - rb
    idx_tc, d_tc_part = pl.pallas_call(
        _knn_body,
        grid=(B, n_tc // _TQ),
        in_specs=[
            pl.BlockSpec((1, _TQ, d_s), lambda b, t: (b, t, 0)),
            pl.BlockSpec((1, d_s, K), lambda b, t: (b, 0, 0)),
        ],
        out_specs=[
            pl.BlockSpec((1, _TQ, k), lambda b, t: (b, t, 0)),
            pl.BlockSpec((1, _TQ, k), lambda b, t: (b, t, 0)),
        ],
        out_shape=[
            jax.ShapeDtypeStruct((B, n_tc, k), jnp.int32),
            jax.ShapeDtypeStruct((B, n_tc, k), jnp.float32),
        ],
    )(Rq[:, rb:], txT)

    e0 = jnp.broadcast_to(embed_obs[0], (B, Q, embed_obs.shape[1]))
    e1 = jnp.broadcast_to(embed_obs[1], (B, K, embed_obs.shape[1]))
    f_test = jnp.zeros((B, Q, d_f), f_ctx.dtype)
    X = jnp.concatenate([
        jnp.concatenate([e0, s_test, f_test], axis=-1),
        jnp.concatenate([e1, s_ctx, f_ctx], axis=-1),
    ], axis=1).reshape(B * n_rows, -1)
    mlp_rows = 1024
    full = lambda a: pl.BlockSpec(a.shape, lambda t: (0,) * a.ndim)
    x_all = pl.pallas_call(
        _mlp_body,
        grid=(B * n_rows // mlp_rows,),
        in_specs=[
            pl.BlockSpec((mlp_rows, X.shape[-1]), lambda t: (t, 0)),
            full(W1), full(b1.reshape(1, -1)), full(W2),
            full(b2.reshape(1, -1)), full(W3), full(b3.reshape(1, -1)),
            full(ln_scale.reshape(1, -1)), full(ln_bias.reshape(1, -1)),
        ],
        out_specs=pl.BlockSpec((mlp_rows, 64), lambda t: (t, 0)),
        out_shape=jax.ShapeDtypeStruct((B * n_rows, 64), jnp.float32),
    )(X, W1, b1.reshape(1, -1), W2, b2.reshape(1, -1), W3,
      b3.reshape(1, -1), ln_scale.reshape(1, -1),
      ln_bias.reshape(1, -1)).reshape(B, n_rows, 64)

    idx_all = jnp.concatenate([idx_sc, idx_tc], axis=1)
    d_all = jnp.concatenate([d_sc, d_tc_part], axis=1)

    nodes_tc = x_all
    nodes_cc = x_all[:, Q:]
    tx_tc = idx_all[:, :Q].reshape(B, Q * k)
    tx_cc = idx_all[:, Q:].reshape(B, K * k)
    d_tc = d_all[:, :Q].reshape(B, Q * k)
    d_cc = d_all[:, Q:].reshape(B, K * k)
    rx_tc = jnp.broadcast_to(jnp.repeat(jnp.arange(Q), k), (B, Q * k))
    rx_cc = jnp.broadcast_to(jnp.repeat(jnp.arange(K), k), (B, K * k))
    return (nodes_tc, d_tc, rx_tc, Q + tx_tc, nodes_cc, d_cc, rx_cc, tx_cc)


# speedup vs baseline: 3.1007x; 3.1007x over previous
import jax
import jax.numpy as jnp
from jax import lax
from jax.experimental import pallas as pl
from jax.experimental.pallas import tpu as pltpu
from jax.experimental.pallas import tpu_sc as plsc

_K_NN = 10
_TQ = 256
_RB_SC = 2048
_NC, _NS = 2, 16
_NW = _NC * _NS
_CAP = 96
_NVB = _CAP // 16
_SHRINK_AT = 48


def _mlp_body(x_ref, w1_ref, b1_ref, w2_ref, b2_ref, w3_ref, b3_ref,
              lns_ref, lnb_ref, out_ref):
    h = jax.nn.gelu(jnp.dot(x_ref[:], w1_ref[:]) + b1_ref[:])
    h = jax.nn.gelu(jnp.dot(h, w2_ref[:]) + b2_ref[:])
    h = jnp.dot(h, w3_ref[:]) + b3_ref[:]
    mu = jnp.mean(h, axis=-1, keepdims=True)
    var = jnp.var(h, axis=-1, keepdims=True)
    out_ref[:] = (h - mu) / jnp.sqrt(var + 1e-6) * lns_ref[:] + lnb_ref[:]


def _knn_body(q_ref, txT_ref, idx_ref, d_ref):
    q = q_ref[0]
    txT = txT_ref[0]
    acc = None
    for d in range(4):
        diff = q[:, d:d + 1] - txT[d:d + 1, :]
        sq = diff * diff
        acc = sq if acc is None else acc + sq

    n_ctx = acc.shape[1]
    colf = lax.broadcasted_iota(jnp.int32, acc.shape, 1).astype(jnp.float32)
    idx_cols, d_cols = [], []
    d2m = acc
    eqmask = None
    for _ in range(_K_NN):
        if eqmask is not None:
            d2m = jnp.where(eqmask, jnp.inf, d2m)
        m = jnp.min(d2m, axis=1, keepdims=True)
        eqmask = d2m == m
        posf = jnp.min(jnp.where(eqmask, colf, float(n_ctx)), axis=1,
                       keepdims=True)
        idx_cols.append(posf.astype(jnp.int32))
        d_cols.append(jnp.sqrt(jnp.maximum(m, 0.0)))
    idx_ref[0] = jnp.concatenate(idx_cols, axis=1)
    d_ref[0] = jnp.concatenate(d_cols, axis=1)


def _lane_min(v, iota16):
    for s in (1, 2, 4, 8):
        perm = jnp.bitwise_xor(iota16, s)
        v = jnp.minimum(v, v.at[perm].get(mode="promise_in_bounds"))
    return v


def _sc_knn_body(K, rows_w, qc_ref, txT_ref, idx_out, d2_out,
                 txv, qv, outi, outd, bufd, bufi):
    wid = lax.axis_index("s") * _NC + lax.axis_index("c")
    wpb = _NW // 4
    b = wid // wpb
    r0 = (wid % wpb) * rows_w
    pltpu.sync_copy(txT_ref.at[pl.ds(b * 4 * K, 4 * K)], txv)
    pltpu.sync_copy(qc_ref.at[pl.ds((b * _RB_SC + r0) * 4, rows_w * 4)],
                    qv.at[pl.ds(0, rows_w * 4)])
    iota16 = lax.iota(jnp.int32, 16)
    inf_vec = jnp.full((16,), jnp.inf, jnp.float32)
    bigi_vec = jnp.full((16,), 2 ** 30, jnp.int32)

    def row_fn(r, _):
        qvec = qv[pl.ds(r * 4, 16)]
        q0, q1, q2, q3 = qvec[0], qvec[1], qvec[2], qvec[3]
        for i in range(_NVB):
            bufd[pl.ds(i * 16, 16)] = inf_vec

        def _dist(off):
            x0 = txv[pl.ds(off, 16)]
            x1 = txv[pl.ds(K + off, 16)]
            x2 = txv[pl.ds(2 * K + off, 16)]
            x3 = txv[pl.ds(3 * K + off, 16)]
            e0 = x0 - q0
            acc = e0 * e0
            e1 = x1 - q1
            acc = acc + e1 * e1
            e2 = x2 - q2
            acc = acc + e2 * e2
            e3 = x3 - q3
            return acc + e3 * e3

        def scan_fn(v, carry):
            thresh, cnt = carry
            off = v * 32
            acc_a = _dist(off)
            acc_b = _dist(off + 16)
            m_a = acc_a <= thresh
            m_b = acc_b <= thresh
            na = plsc.all_reduce_population_count(m_a)
            nb = plsc.all_reduce_population_count(m_b)

            def append1(c0, m, acc, ioff):
                mi = m.astype(jnp.int32)
                cs = plsc.cumsum(mi)
                posv = cs - 1 + c0
                msafe = m & (posv < _CAP)
                plsc.store_scatter(bufd, (posv,), acc, mask=msafe)
                plsc.store_scatter(bufi, (posv,), iota16 + ioff, mask=msafe)
                return c0 + cs[15]

            def append(op):
                t0, c0 = op
                c1 = lax.cond(na[0] > 0,
                              lambda c: append1(c, m_a, acc_a, off),
                              lambda c: c, c0)
                c1 = lax.cond(nb[0] > 0,
                              lambda c: append1(c, m_b, acc_b, off + 16),
                              lambda c: c, c1)

                def shrink(op2):
                    _t, c_in = op2
                    vs = [bufd[pl.ds(i * 16, 16)] for i in range(_NVB)]
                    bis = [bufi[pl.ds(i * 16, 16)] for i in range(_NVB)]
                    work = list(vs)
                    t = inf_vec
                    for _ in range(_K_NN):
                        mm = work[0]
                        for i in range(1, _NVB):
                            mm = jnp.minimum(mm, work[i])
                        t = _lane_min(mm, iota16)
                        work = [jnp.where(w == t, inf_vec, w) for w in work]
                    for i in range(_NVB):
                        bufd[pl.ds(i * 16, 16)] = inf_vec
                    c2 = jnp.zeros((16,), jnp.int32)
                    for i in range(_NVB):
                        keep = vs[i] <= t
                        ki = keep.astype(jnp.int32)
                        csk = plsc.cumsum(ki)
                        pk = csk - 1 + c2
                        plsc.store_scatter(bufd, (pk,), vs[i], mask=keep)
                        plsc.store_scatter(bufi, (pk,), bis[i], mask=keep)
                        c2 = c2 + csk[15]
                    return t, c2

                return lax.cond(c1[0] > _SHRINK_AT, shrink, lambda op2: op2,
                                (t0, c1))

            return lax.cond(na[0] + nb[0] > 0, append, lambda op: op,
                            (thresh, cnt))

        lax.fori_loop(0, K // 32, scan_fn,
                      (inf_vec, jnp.zeros((16,), jnp.int32)))

        vs = [bufd[pl.ds(i * 16, 16)] for i in range(_NVB)]
        bis = [bufi[pl.ds(i * 16, 16)] for i in range(_NVB)]
        work = list(vs)
        idxacc = jnp.zeros((16,), jnp.int32)
        dacc = inf_vec
        for kk in range(_K_NN):
            mm = work[0]
            for i in range(1, _NVB):
                mm = jnp.minimum(mm, work[i])
            mval = _lane_min(mm, iota16)
            iv = bigi_vec
            for i in range(_NVB):
                iv = jnp.minimum(iv, jnp.where(work[i] == mval, bis[i],
                                               bigi_vec))
            cidx = _lane_min(iv, iota16)
            work = [jnp.where((work[i] == mval) & (bis[i] == cidx), jnp.inf,
                              work[i]) for i in range(_NVB)]
            sel = iota16 == kk
            idxacc = jnp.where(sel, cidx, idxacc)
            dacc = jnp.where(sel, mval, dacc)
        outi[pl.ds(r * 16, 16)] = idxacc
        outd[pl.ds(r * 16, 16)] = dacc
        return 0

    lax.fori_loop(0, rows_w, row_fn, 0)
    base = (b * _RB_SC + r0) * 16
    pltpu.sync_copy(outi, idx_out.at[pl.ds(base, rows_w * 16)])
    pltpu.sync_copy(outd, d2_out.at[pl.ds(base, rows_w * 16)])


def kernel(s_ctx, f_ctx, s_test, embed_obs, W1, b1, W2, b2, W3, b3,
           ln_scale, ln_bias):
    k = _K_NN
    B, Q, d_s = s_test.shape
    K = s_ctx.shape[1]
    d_f = f_ctx.shape[-1]
    n_rows = Q + K
    rb = _RB_SC
    rows_w = B * rb // _NW

    Rq = jnp.concatenate([s_test, s_ctx], axis=1)
    txT = jnp.swapaxes(s_ctx, 1, 2)

    mesh = plsc.VectorSubcoreMesh(core_axis_name="c", subcore_axis_name="s",
                                  num_cores=_NC, num_subcores=_NS)
    sc_idx, sc_d2 = pl.kernel(
        lambda *refs: _sc_knn_body(K, rows_w, *refs),
        out_type=[
            jax.ShapeDtypeStruct((B * rb * 16,), jnp.int32),
            jax.ShapeDtypeStruct((B * rb * 16,), jnp.float32),
        ],
        mesh=mesh,
        compiler_params=pltpu.CompilerParams(needs_layout_passes=False),
        scratch_types=[
            pltpu.VMEM((4 * K,), jnp.float32),
            pltpu.VMEM((rows_w * 4 + 16,), jnp.float32),
            pltpu.VMEM((rows_w * 16,), jnp.int32),
            pltpu.VMEM((rows_w * 16,), jnp.float32),
            pltpu.VMEM((_CAP,), jnp.float32),
            pltpu.VMEM((_CAP,), jnp.int32),
        ],
    )(Rq[:, :rb].reshape(-1), txT.reshape(-1))
    idx_sc = sc_idx.reshape(B, rb, 16)[..., :k]
    d_sc = jnp.sqrt(jnp.maximum(sc_d2.reshape(B, rb, 16)[..., :k], 0.0))

    n_tc = n_rows - rb
    idx_tc, d_tc_part = pl.pallas_call(
        _knn_body,
        grid=(B, n_tc // _TQ),
        in_specs=[
            pl.BlockSpec((1, _TQ, d_s), lambda b, t: (b, t, 0)),
            pl.BlockSpec((1, d_s, K), lambda b, t: (b, 0, 0)),
        ],
        out_specs=[
            pl.BlockSpec((1, _TQ, k), lambda b, t: (b, t, 0)),
            pl.BlockSpec((1, _TQ, k), lambda b, t: (b, t, 0)),
        ],
        out_shape=[
            jax.ShapeDtypeStruct((B, n_tc, k), jnp.int32),
            jax.ShapeDtypeStruct((B, n_tc, k), jnp.float32),
        ],
    )(Rq[:, rb:], txT)

    e0 = jnp.broadcast_to(embed_obs[0], (B, Q, embed_obs.shape[1]))
    e1 = jnp.broadcast_to(embed_obs[1], (B, K, embed_obs.shape[1]))
    f_test = jnp.zeros((B, Q, d_f), f_ctx.dtype)
    X = jnp.concatenate([
        jnp.concatenate([e0, s_test, f_test], axis=-1),
        jnp.concatenate([e1, s_ctx, f_ctx], axis=-1),
    ], axis=1).reshape(B * n_rows, -1)
    mlp_rows = 1024
    full = lambda a: pl.BlockSpec(a.shape, lambda t: (0,) * a.ndim)
    x_all = pl.pallas_call(
        _mlp_body,
        grid=(B * n_rows // mlp_rows,),
        in_specs=[
            pl.BlockSpec((mlp_rows, X.shape[-1]), lambda t: (t, 0)),
            full(W1), full(b1.reshape(1, -1)), full(W2),
            full(b2.reshape(1, -1)), full(W3), full(b3.reshape(1, -1)),
            full(ln_scale.reshape(1, -1)), full(ln_bias.reshape(1, -1)),
        ],
        out_specs=pl.BlockSpec((mlp_rows, 64), lambda t: (t, 0)),
        out_shape=jax.ShapeDtypeStruct((B * n_rows, 64), jnp.float32),
    )(X, W1, b1.reshape(1, -1), W2, b2.reshape(1, -1), W3,
      b3.reshape(1, -1), ln_scale.reshape(1, -1),
      ln_bias.reshape(1, -1)).reshape(B, n_rows, 64)

    idx_all = jnp.concatenate([idx_sc, idx_tc], axis=1)
    d_all = jnp.concatenate([d_sc, d_tc_part], axis=1)

    nodes_tc = x_all
    nodes_cc = x_all[:, Q:]
    tx_tc = idx_all[:, :Q].reshape(B, Q * k)
    tx_cc = idx_all[:, Q:].reshape(B, K * k)
    d_tc = d_all[:, :Q].reshape(B, Q * k)
    d_cc = d_all[:, Q:].reshape(B, K * k)
    rx_tc = jnp.broadcast_to(jnp.repeat(jnp.arange(Q), k), (B, Q * k))
    rx_cc = jnp.broadcast_to(jnp.repeat(jnp.arange(K), k), (B, K * k))
    return (nodes_tc, d_tc, rx_tc, Q + tx_tc, nodes_cc, d_cc, rx_cc, tx_cc)


# speedup vs baseline: 5.9393x; 1.9155x over previous
import jax
import jax.numpy as jnp
from jax import lax
from jax.experimental import pallas as pl
from jax.experimental.pallas import tpu as pltpu
from jax.experimental.pallas import tpu_sc as plsc

_K_NN = 10
_TQ = 256
_RB_SC = 1024
_NC, _NS = 2, 16
_NW = _NC * _NS
_CAP = 96
_NVB = _CAP // 16
_SHRINK_AT = 48


def _mlp_body(x_ref, w1_ref, b1_ref, w2_ref, b2_ref, w3_ref, b3_ref,
              lns_ref, lnb_ref, out_ref):
    h = jax.nn.gelu(jnp.dot(x_ref[:], w1_ref[:]) + b1_ref[:])
    h = jax.nn.gelu(jnp.dot(h, w2_ref[:]) + b2_ref[:])
    h = jnp.dot(h, w3_ref[:]) + b3_ref[:]
    mu = jnp.mean(h, axis=-1, keepdims=True)
    var = jnp.var(h, axis=-1, keepdims=True)
    out_ref[:] = (h - mu) / jnp.sqrt(var + 1e-6) * lns_ref[:] + lnb_ref[:]


def _knn_body(q_ref, txT_ref, idx_ref, d_ref):
    q = q_ref[0]
    txT = txT_ref[0]
    acc = None
    for d in range(4):
        diff = q[:, d:d + 1] - txT[d:d + 1, :]
        sq = diff * diff
        acc = sq if acc is None else acc + sq

    n_ctx = acc.shape[1]
    colf = lax.broadcasted_iota(jnp.int32, acc.shape, 1).astype(jnp.float32)
    idx_cols, d_cols = [], []
    d2m = acc
    eqmask = None
    for _ in range(_K_NN):
        if eqmask is not None:
            d2m = jnp.where(eqmask, jnp.inf, d2m)
        m = jnp.min(d2m, axis=1, keepdims=True)
        eqmask = d2m == m
        posf = jnp.min(jnp.where(eqmask, colf, float(n_ctx)), axis=1,
                       keepdims=True)
        idx_cols.append(posf.astype(jnp.int32))
        d_cols.append(jnp.sqrt(jnp.maximum(m, 0.0)))
    idx_ref[0] = jnp.concatenate(idx_cols, axis=1)
    d_ref[0] = jnp.concatenate(d_cols, axis=1)


def _lane_min(v, iota16):
    for s in (1, 2, 4, 8):
        perm = jnp.bitwise_xor(iota16, s)
        v = jnp.minimum(v, v.at[perm].get(mode="promise_in_bounds"))
    return v


def _sc_knn_body(K, rows_w, qc_ref, txT_ref, idx_out, d2_out,
                 txv, qv, outi, outd, bufd, bufi):
    wid = lax.axis_index("s") * _NC + lax.axis_index("c")
    wpb = _NW // 4
    b = wid // wpb
    r0 = (wid % wpb) * rows_w
    pltpu.sync_copy(txT_ref.at[pl.ds(b * 4 * K, 4 * K)], txv)
    pltpu.sync_copy(qc_ref.at[pl.ds((b * _RB_SC + r0) * 4, rows_w * 4)],
                    qv.at[pl.ds(0, rows_w * 4)])
    iota16 = lax.iota(jnp.int32, 16)
    inf_vec = jnp.full((16,), jnp.inf, jnp.float32)
    bigi_vec = jnp.full((16,), 2 ** 30, jnp.int32)

    def row_fn(r, _):
        qvec = qv[pl.ds(r * 4, 16)]
        q0, q1, q2, q3 = qvec[0], qvec[1], qvec[2], qvec[3]
        for i in range(_NVB):
            bufd[pl.ds(i * 16, 16)] = inf_vec

        def _dist(off):
            x0 = txv[pl.ds(off, 16)]
            x1 = txv[pl.ds(K + off, 16)]
            x2 = txv[pl.ds(2 * K + off, 16)]
            x3 = txv[pl.ds(3 * K + off, 16)]
            e0 = x0 - q0
            acc = e0 * e0
            e1 = x1 - q1
            acc = acc + e1 * e1
            e2 = x2 - q2
            acc = acc + e2 * e2
            e3 = x3 - q3
            return acc + e3 * e3

        def scan_fn(v, carry):
            thresh, cnt = carry
            off = v * 32
            acc_a = _dist(off)
            acc_b = _dist(off + 16)
            m_a = acc_a <= thresh
            m_b = acc_b <= thresh
            na = plsc.all_reduce_population_count(m_a)
            nb = plsc.all_reduce_population_count(m_b)

            def append1(c0, m, acc, ioff):
                mi = m.astype(jnp.int32)
                cs = plsc.cumsum(mi)
                posv = cs - 1 + c0
                msafe = m & (posv < _CAP)
                plsc.store_scatter(bufd, (posv,), acc, mask=msafe)
                plsc.store_scatter(bufi, (posv,), iota16 + ioff, mask=msafe)
                return c0 + cs[15]

            def append(op):
                t0, c0 = op
                c1 = lax.cond(na[0] > 0,
                              lambda c: append1(c, m_a, acc_a, off),
                              lambda c: c, c0)
                c1 = lax.cond(nb[0] > 0,
                              lambda c: append1(c, m_b, acc_b, off + 16),
                              lambda c: c, c1)

                def shrink(op2):
                    _t, c_in = op2
                    vs = [bufd[pl.ds(i * 16, 16)] for i in range(_NVB)]
                    bis = [bufi[pl.ds(i * 16, 16)] for i in range(_NVB)]
                    work = list(vs)
                    t = inf_vec
                    for _ in range(_K_NN):
                        mm = work[0]
                        for i in range(1, _NVB):
                            mm = jnp.minimum(mm, work[i])
                        t = _lane_min(mm, iota16)
                        work = [jnp.where(w == t, inf_vec, w) for w in work]
                    for i in range(_NVB):
                        bufd[pl.ds(i * 16, 16)] = inf_vec
                    c2 = jnp.zeros((16,), jnp.int32)
                    for i in range(_NVB):
                        keep = vs[i] <= t
                        ki = keep.astype(jnp.int32)
                        csk = plsc.cumsum(ki)
                        pk = csk - 1 + c2
                        plsc.store_scatter(bufd, (pk,), vs[i], mask=keep)
                        plsc.store_scatter(bufi, (pk,), bis[i], mask=keep)
                        c2 = c2 + csk[15]
                    return t, c2

                return lax.cond(c1[0] > _SHRINK_AT, shrink, lambda op2: op2,
                                (t0, c1))

            return lax.cond(na[0] + nb[0] > 0, append, lambda op: op,
                            (thresh, cnt))

        lax.fori_loop(0, K // 32, scan_fn,
                      (inf_vec, jnp.zeros((16,), jnp.int32)))

        vs = [bufd[pl.ds(i * 16, 16)] for i in range(_NVB)]
        bis = [bufi[pl.ds(i * 16, 16)] for i in range(_NVB)]
        work = list(vs)
        idxacc = jnp.zeros((16,), jnp.int32)
        dacc = inf_vec
        for kk in range(_K_NN):
            mm = work[0]
            for i in range(1, _NVB):
                mm = jnp.minimum(mm, work[i])
            mval = _lane_min(mm, iota16)
            iv = bigi_vec
            for i in range(_NVB):
                iv = jnp.minimum(iv, jnp.where(work[i] == mval, bis[i],
                                               bigi_vec))
            cidx = _lane_min(iv, iota16)
            work = [jnp.where((work[i] == mval) & (bis[i] == cidx), jnp.inf,
                              work[i]) for i in range(_NVB)]
            sel = iota16 == kk
            idxacc = jnp.where(sel, cidx, idxacc)
            dacc = jnp.where(sel, mval, dacc)
        outi[pl.ds(r * 16, 16)] = idxacc
        outd[pl.ds(r * 16, 16)] = dacc
        return 0

    lax.fori_loop(0, rows_w, row_fn, 0)
    base = (b * _RB_SC + r0) * 16
    pltpu.sync_copy(outi, idx_out.at[pl.ds(base, rows_w * 16)])
    pltpu.sync_copy(outd, d2_out.at[pl.ds(base, rows_w * 16)])


def kernel(s_ctx, f_ctx, s_test, embed_obs, W1, b1, W2, b2, W3, b3,
           ln_scale, ln_bias):
    k = _K_NN
    B, Q, d_s = s_test.shape
    K = s_ctx.shape[1]
    d_f = f_ctx.shape[-1]
    n_rows = Q + K
    rb = _RB_SC
    rows_w = B * rb // _NW

    Rq = jnp.concatenate([s_test, s_ctx], axis=1)
    txT = jnp.swapaxes(s_ctx, 1, 2)

    mesh = plsc.VectorSubcoreMesh(core_axis_name="c", subcore_axis_name="s",
                                  num_cores=_NC, num_subcores=_NS)
    sc_idx, sc_d2 = pl.kernel(
        lambda *refs: _sc_knn_body(K, rows_w, *refs),
        out_type=[
            jax.ShapeDtypeStruct((B * rb * 16,), jnp.int32),
            jax.ShapeDtypeStruct((B * rb * 16,), jnp.float32),
        ],
        mesh=mesh,
        compiler_params=pltpu.CompilerParams(needs_layout_passes=False),
        scratch_types=[
            pltpu.VMEM((4 * K,), jnp.float32),
            pltpu.VMEM((rows_w * 4 + 16,), jnp.float32),
            pltpu.VMEM((rows_w * 16,), jnp.int32),
            pltpu.VMEM((rows_w * 16,), jnp.float32),
            pltpu.VMEM((_CAP,), jnp.float32),
            pltpu.VMEM((_CAP,), jnp.int32),
        ],
    )(Rq[:, :rb].reshape(-1), txT.reshape(-1))
    idx_sc = sc_idx.reshape(B, rb, 16)[..., :k]
    d_sc = jnp.sqrt(jnp.maximum(sc_d2.reshape(B, rb, 16)[..., :k], 0.0))

    n_tc = n_rows - rb
    idx_tc, d_tc_part = pl.pallas_call(
        _knn_body,
        grid=(B, n_tc // _TQ),
        in_specs=[
            pl.BlockSpec((1, _TQ, d_s), lambda b, t: (b, t, 0)),
            pl.BlockSpec((1, d_s, K), lambda b, t: (b, 0, 0)),
        ],
        out_specs=[
            pl.BlockSpec((1, _TQ, k), lambda b, t: (b, t, 0)),
            pl.BlockSpec((1, _TQ, k), lambda b, t: (b, t, 0)),
        ],
        out_shape=[
            jax.ShapeDtypeStruct((B, n_tc, k), jnp.int32),
            jax.ShapeDtypeStruct((B, n_tc, k), jnp.float32),
        ],
    )(Rq[:, rb:], txT)

    e0 = jnp.broadcast_to(embed_obs[0], (B, Q, embed_obs.shape[1]))
    e1 = jnp.broadcast_to(embed_obs[1], (B, K, embed_obs.shape[1]))
    f_test = jnp.zeros((B, Q, d_f), f_ctx.dtype)
    X = jnp.concatenate([
        jnp.concatenate([e0, s_test, f_test], axis=-1),
        jnp.concatenate([e1, s_ctx, f_ctx], axis=-1),
    ], axis=1).reshape(B * n_rows, -1)
    mlp_rows = 1024
    full = lambda a: pl.BlockSpec(a.shape, lambda t: (0,) * a.ndim)
    x_all = pl.pallas_call(
        _mlp_body,
        grid=(B * n_rows // mlp_rows,),
        in_specs=[
            pl.BlockSpec((mlp_rows, X.shape[-1]), lambda t: (t, 0)),
            full(W1), full(b1.reshape(1, -1)), full(W2),
            full(b2.reshape(1, -1)), full(W3), full(b3.reshape(1, -1)),
            full(ln_scale.reshape(1, -1)), full(ln_bias.reshape(1, -1)),
        ],
        out_specs=pl.BlockSpec((mlp_rows, 64), lambda t: (t, 0)),
        out_shape=jax.ShapeDtypeStruct((B * n_rows, 64), jnp.float32),
    )(X, W1, b1.reshape(1, -1), W2, b2.reshape(1, -1), W3,
      b3.reshape(1, -1), ln_scale.reshape(1, -1),
      ln_bias.reshape(1, -1)).reshape(B, n_rows, 64)

    idx_all = jnp.concatenate([idx_sc, idx_tc], axis=1)
    d_all = jnp.concatenate([d_sc, d_tc_part], axis=1)

    nodes_tc = x_all
    nodes_cc = x_all[:, Q:]
    tx_tc = idx_all[:, :Q].reshape(B, Q * k)
    tx_cc = idx_all[:, Q:].reshape(B, K * k)
    d_tc = d_all[:, :Q].reshape(B, Q * k)
    d_cc = d_all[:, Q:].reshape(B, K * k)
    rx_tc = jnp.broadcast_to(jnp.repeat(jnp.arange(Q), k), (B, Q * k))
    rx_cc = jnp.broadcast_to(jnp.repeat(jnp.arange(K), k), (B, K * k))
    return (nodes_tc, d_tc, rx_tc, Q + tx_tc, nodes_cc, d_cc, rx_cc, tx_cc)


# speedup vs baseline: 6.7824x; 1.1420x over previous
import jax
import jax.numpy as jnp
from jax import lax
from jax.experimental import pallas as pl
from jax.experimental.pallas import tpu as pltpu
from jax.experimental.pallas import tpu_sc as plsc

_K_NN = 10
_TQ = 256
_RB_SC = 1280
_NC, _NS = 2, 16
_NW = _NC * _NS
_CAP = 96
_NVB = _CAP // 16
_SHRINK_AT = 32


def _mlp_body(x_ref, w1_ref, b1_ref, w2_ref, b2_ref, w3_ref, b3_ref,
              lns_ref, lnb_ref, out_ref):
    h = jax.nn.gelu(jnp.dot(x_ref[:], w1_ref[:]) + b1_ref[:])
    h = jax.nn.gelu(jnp.dot(h, w2_ref[:]) + b2_ref[:])
    h = jnp.dot(h, w3_ref[:]) + b3_ref[:]
    mu = jnp.mean(h, axis=-1, keepdims=True)
    var = jnp.var(h, axis=-1, keepdims=True)
    out_ref[:] = (h - mu) / jnp.sqrt(var + 1e-6) * lns_ref[:] + lnb_ref[:]


def _knn_body(q_ref, txT_ref, idx_ref, d_ref):
    q = q_ref[0]
    txT = txT_ref[0]
    acc = None
    for d in range(4):
        diff = q[:, d:d + 1] - txT[d:d + 1, :]
        sq = diff * diff
        acc = sq if acc is None else acc + sq

    n_ctx = acc.shape[1]
    colf = lax.broadcasted_iota(jnp.int32, acc.shape, 1).astype(jnp.float32)
    idx_cols, d_cols = [], []
    d2m = acc
    eqmask = None
    for _ in range(_K_NN):
        if eqmask is not None:
            d2m = jnp.where(eqmask, jnp.inf, d2m)
        m = jnp.min(d2m, axis=1, keepdims=True)
        eqmask = d2m == m
        posf = jnp.min(jnp.where(eqmask, colf, float(n_ctx)), axis=1,
                       keepdims=True)
        idx_cols.append(posf.astype(jnp.int32))
        d_cols.append(jnp.sqrt(jnp.maximum(m, 0.0)))
    idx_ref[0] = jnp.concatenate(idx_cols, axis=1)
    d_ref[0] = jnp.concatenate(d_cols, axis=1)


def _lane_min(v, iota16):
    for s in (1, 2, 4, 8):
        perm = jnp.bitwise_xor(iota16, s)
        v = jnp.minimum(v, v.at[perm].get(mode="promise_in_bounds"))
    return v


def _sc_knn_body(K, rows_w, qc_ref, txT_ref, idx_out, d2_out,
                 txv, qv, outi, outd, bufd, bufi):
    wid = lax.axis_index("s") * _NC + lax.axis_index("c")
    wpb = _NW // 4
    b = wid // wpb
    r0 = (wid % wpb) * rows_w
    pltpu.sync_copy(txT_ref.at[pl.ds(b * 4 * K, 4 * K)], txv)
    pltpu.sync_copy(qc_ref.at[pl.ds((b * _RB_SC + r0) * 4, rows_w * 4)],
                    qv.at[pl.ds(0, rows_w * 4)])
    iota16 = lax.iota(jnp.int32, 16)
    inf_vec = jnp.full((16,), jnp.inf, jnp.float32)
    bigi_vec = jnp.full((16,), 2 ** 30, jnp.int32)

    def row_fn(r, _):
        qvec = qv[pl.ds(r * 4, 16)]
        q0, q1, q2, q3 = qvec[0], qvec[1], qvec[2], qvec[3]
        for i in range(_NVB):
            bufd[pl.ds(i * 16, 16)] = inf_vec

        def _dist(off):
            x0 = txv[pl.ds(off, 16)]
            x1 = txv[pl.ds(K + off, 16)]
            x2 = txv[pl.ds(2 * K + off, 16)]
            x3 = txv[pl.ds(3 * K + off, 16)]
            e0 = x0 - q0
            acc = e0 * e0
            e1 = x1 - q1
            acc = acc + e1 * e1
            e2 = x2 - q2
            acc = acc + e2 * e2
            e3 = x3 - q3
            return acc + e3 * e3

        def scan_fn(v, carry):
            thresh, cnt = carry
            off = v * 64
            offs = [off, off + 16, off + 32, off + 48]
            accs = [_dist(o) for o in offs]
            ms = [a <= thresh for a in accs]
            ns = [plsc.all_reduce_population_count(m) for m in ms]

            def append1(c0, m, acc, ioff):
                mi = m.astype(jnp.int32)
                cs = plsc.cumsum(mi)
                posv = cs - 1 + c0
                msafe = m & (posv < _CAP)
                plsc.store_scatter(bufd, (posv,), acc, mask=msafe)
                plsc.store_scatter(bufi, (posv,), iota16 + ioff, mask=msafe)
                return c0 + cs[15]

            def append(op):
                t0, c0 = op
                c1 = c0
                for j in range(4):
                    c1 = lax.cond(ns[j][0] > 0,
                                  lambda c, j=j: append1(c, ms[j], accs[j],
                                                         offs[j]),
                                  lambda c: c, c1)

                def shrink(op2):
                    _t, c_in = op2
                    vs = [bufd[pl.ds(i * 16, 16)] for i in range(_NVB)]
                    bis = [bufi[pl.ds(i * 16, 16)] for i in range(_NVB)]
                    work = list(vs)
                    t = inf_vec
                    for _ in range(_K_NN):
                        mm = work[0]
                        for i in range(1, _NVB):
                            mm = jnp.minimum(mm, work[i])
                        t = _lane_min(mm, iota16)
                        work = [jnp.where(w == t, inf_vec, w) for w in work]
                    for i in range(_NVB):
                        bufd[pl.ds(i * 16, 16)] = inf_vec
                    c2 = jnp.zeros((16,), jnp.int32)
                    for i in range(_NVB):
                        keep = vs[i] <= t
                        ki = keep.astype(jnp.int32)
                        csk = plsc.cumsum(ki)
                        pk = csk - 1 + c2
                        plsc.store_scatter(bufd, (pk,), vs[i], mask=keep)
                        plsc.store_scatter(bufi, (pk,), bis[i], mask=keep)
                        c2 = c2 + csk[15]
                    return t, c2

                return lax.cond(c1[0] > _SHRINK_AT, shrink, lambda op2: op2,
                                (t0, c1))

            nsum = ns[0][0] + ns[1][0] + ns[2][0] + ns[3][0]
            return lax.cond(nsum > 0, append, lambda op: op,
                            (thresh, cnt))

        lax.fori_loop(0, K // 64, scan_fn,
                      (inf_vec, jnp.zeros((16,), jnp.int32)))

        vs = [bufd[pl.ds(i * 16, 16)] for i in range(_NVB)]
        bis = [bufi[pl.ds(i * 16, 16)] for i in range(_NVB)]
        work = list(vs)
        idxacc = jnp.zeros((16,), jnp.int32)
        dacc = inf_vec
        for kk in range(_K_NN):
            mm = work[0]
            for i in range(1, _NVB):
                mm = jnp.minimum(mm, work[i])
            mval = _lane_min(mm, iota16)
            iv = bigi_vec
            for i in range(_NVB):
                iv = jnp.minimum(iv, jnp.where(work[i] == mval, bis[i],
                                               bigi_vec))
            cidx = _lane_min(iv, iota16)
            work = [jnp.where((work[i] == mval) & (bis[i] == cidx), jnp.inf,
                              work[i]) for i in range(_NVB)]
            sel = iota16 == kk
            idxacc = jnp.where(sel, cidx, idxacc)
            dacc = jnp.where(sel, mval, dacc)
        outi[pl.ds(r * 16, 16)] = idxacc
        outd[pl.ds(r * 16, 16)] = dacc
        return 0

    lax.fori_loop(0, rows_w, row_fn, 0)
    base = (b * _RB_SC + r0) * 16
    pltpu.sync_copy(outi, idx_out.at[pl.ds(base, rows_w * 16)])
    pltpu.sync_copy(outd, d2_out.at[pl.ds(base, rows_w * 16)])


def kernel(s_ctx, f_ctx, s_test, embed_obs, W1, b1, W2, b2, W3, b3,
           ln_scale, ln_bias):
    k = _K_NN
    B, Q, d_s = s_test.shape
    K = s_ctx.shape[1]
    d_f = f_ctx.shape[-1]
    n_rows = Q + K
    rb = _RB_SC
    rows_w = B * rb // _NW

    Rq = jnp.concatenate([s_test, s_ctx], axis=1)
    txT = jnp.swapaxes(s_ctx, 1, 2)

    mesh = plsc.VectorSubcoreMesh(core_axis_name="c", subcore_axis_name="s",
                                  num_cores=_NC, num_subcores=_NS)
    sc_idx, sc_d2 = pl.kernel(
        lambda *refs: _sc_knn_body(K, rows_w, *refs),
        out_type=[
            jax.ShapeDtypeStruct((B * rb * 16,), jnp.int32),
            jax.ShapeDtypeStruct((B * rb * 16,), jnp.float32),
        ],
        mesh=mesh,
        compiler_params=pltpu.CompilerParams(needs_layout_passes=False),
        scratch_types=[
            pltpu.VMEM((4 * K,), jnp.float32),
            pltpu.VMEM((rows_w * 4 + 16,), jnp.float32),
            pltpu.VMEM((rows_w * 16,), jnp.int32),
            pltpu.VMEM((rows_w * 16,), jnp.float32),
            pltpu.VMEM((_CAP,), jnp.float32),
            pltpu.VMEM((_CAP,), jnp.int32),
        ],
    )(Rq[:, :rb].reshape(-1), txT.reshape(-1))
    idx_sc = sc_idx.reshape(B, rb, 16)[..., :k]
    d_sc = jnp.sqrt(jnp.maximum(sc_d2.reshape(B, rb, 16)[..., :k], 0.0))

    n_tc = n_rows - rb
    idx_tc, d_tc_part = pl.pallas_call(
        _knn_body,
        grid=(B, n_tc // _TQ),
        in_specs=[
            pl.BlockSpec((1, _TQ, d_s), lambda b, t: (b, t, 0)),
            pl.BlockSpec((1, d_s, K), lambda b, t: (b, 0, 0)),
        ],
        out_specs=[
            pl.BlockSpec((1, _TQ, k), lambda b, t: (b, t, 0)),
            pl.BlockSpec((1, _TQ, k), lambda b, t: (b, t, 0)),
        ],
        out_shape=[
            jax.ShapeDtypeStruct((B, n_tc, k), jnp.int32),
            jax.ShapeDtypeStruct((B, n_tc, k), jnp.float32),
        ],
    )(Rq[:, rb:], txT)

    e0 = jnp.broadcast_to(embed_obs[0], (B, Q, embed_obs.shape[1]))
    e1 = jnp.broadcast_to(embed_obs[1], (B, K, embed_obs.shape[1]))
    f_test = jnp.zeros((B, Q, d_f), f_ctx.dtype)
    X = jnp.concatenate([
        jnp.concatenate([e0, s_test, f_test], axis=-1),
        jnp.concatenate([e1, s_ctx, f_ctx], axis=-1),
    ], axis=1).reshape(B * n_rows, -1)
    mlp_rows = 1024
    full = lambda a: pl.BlockSpec(a.shape, lambda t: (0,) * a.ndim)
    x_all = pl.pallas_call(
        _mlp_body,
        grid=(B * n_rows // mlp_rows,),
        in_specs=[
            pl.BlockSpec((mlp_rows, X.shape[-1]), lambda t: (t, 0)),
            full(W1), full(b1.reshape(1, -1)), full(W2),
            full(b2.reshape(1, -1)), full(W3), full(b3.reshape(1, -1)),
            full(ln_scale.reshape(1, -1)), full(ln_bias.reshape(1, -1)),
        ],
        out_specs=pl.BlockSpec((mlp_rows, 64), lambda t: (t, 0)),
        out_shape=jax.ShapeDtypeStruct((B * n_rows, 64), jnp.float32),
    )(X, W1, b1.reshape(1, -1), W2, b2.reshape(1, -1), W3,
      b3.reshape(1, -1), ln_scale.reshape(1, -1),
      ln_bias.reshape(1, -1)).reshape(B, n_rows, 64)

    idx_all = jnp.concatenate([idx_sc, idx_tc], axis=1)
    d_all = jnp.concatenate([d_sc, d_tc_part], axis=1)

    nodes_tc = x_all
    nodes_cc = x_all[:, Q:]
    tx_tc = idx_all[:, :Q].reshape(B, Q * k)
    tx_cc = idx_all[:, Q:].reshape(B, K * k)
    d_tc = d_all[:, :Q].reshape(B, Q * k)
    d_cc = d_all[:, Q:].reshape(B, K * k)
    rx_tc = jnp.broadcast_to(jnp.repeat(jnp.arange(Q), k), (B, Q * k))
    rx_cc = jnp.broadcast_to(jnp.repeat(jnp.arange(K), k), (B, K * k))
    return (nodes_tc, d_tc, rx_tc, Q + tx_tc, nodes_cc, d_cc, rx_cc, tx_cc)
